# baseline (speedup 1.0000x reference)
import functools

import jax
import jax.numpy as jnp
from jax import lax
from jax.experimental import pallas as pl
from jax.experimental.pallas import tpu as pltpu
from jax.experimental.pallas import tpu_sc as plsc

_N = 16384
_D = 32
_V = 1000000
_NC = 2
_NS = 16
_NW = _NC * _NS
_L = 16
_CW = 1024
_NFULL = _V // _CW
_TAILW = _V - _NFULL * _CW
_TAILP = 640
_NCHUNKS = _NFULL + 1
_HCAP = 4096 + _L
_SCAP = 1536
_AROWS = _N // 4


def _filter_range(idx_v, hits_i, hits_n, lo_col, hi_col, max_row):
    iota = lax.iota(jnp.int32, _L)

    def body(k, pos):
        iv = jnp.minimum(idx_v[pl.ds(k * _L, _L)], max_row)
        nv = iota + k * _L
        m = (iv >= lo_col) & (iv < hi_col)
        mi = m.astype(jnp.int32)
        pfx = plsc.cumsum(mi)
        dest = pos + pfx - mi
        ms = m & (dest < _HCAP - _L)
        plsc.store_scatter(hits_i, [dest], iv, mask=ms)
        plsc.store_scatter(hits_n, [dest], nv, mask=ms)
        return pos + plsc.all_reduce_population_count(m)[0]

    return lax.fori_loop(0, _N // _L, body, jnp.int32(0), unroll=8)


def _make_process(acc, subl_i, subl_n, hits_i, hits_n, staging, staging_b,
                  sem_a, sem_b, count):
    iota = lax.iota(jnp.int32, _L)

    def process(buf, c0, cw):
        nh = (count + _L - 1) // _L

        def sub_pass(r):
            base = r * _SCAP

            def sub_body(j, spos):
                valid = iota < (count - j * _L)
                iv = hits_i[pl.ds(j * _L, _L)]
                nv = hits_n[pl.ds(j * _L, _L)]
                m = valid & (iv >= c0) & (iv < c0 + cw)
                mi = m.astype(jnp.int32)
                pfx = plsc.cumsum(mi)
                dest = spos + pfx - mi - base
                ms = m & (dest >= 0) & (dest < _SCAP)
                dclip = jnp.where(ms, dest, 0)
                plsc.store_scatter(subl_i, [dclip], iv, mask=ms)
                plsc.store_scatter(subl_n, [dclip], nv, mask=ms)
                return spos + plsc.all_reduce_population_count(m)[0]

            return lax.fori_loop(0, nh, sub_body, jnp.int32(0))

        def extract(nsel):
            ngrp = (nsel + _L - 1) // _L

            def one_group(g, stg, sem):
                m = iota < (nsel - g * _L)
                iv = subl_i[pl.ds(g * _L, _L)]
                nv = subl_n[pl.ds(g * _L, _L)]
                colv = jnp.where(m, iv - c0, 0)
                seg = jnp.where(m, (nv & 3) << 5, 0)
                jrow = jnp.where(m, nv >> 2, 0)

                @pl.when(g >= 2)
                def _():
                    pltpu.make_async_copy(stg, acc.at[iota], sem).wait()

                zero = jnp.zeros((_L,), jnp.float32)
                for r in range(_L):
                    for q in range(128 // _L):
                        stg[r, pl.ds(q * _L, _L)] = zero
                for d in range(_D):
                    vals = plsc.load_gather(
                        buf, [jnp.full((_L,), d, jnp.int32), colv], mask=m)
                    plsc.store_scatter(stg, [iota, seg + d], vals, mask=m)
                pltpu.async_copy(stg, acc.at[jrow], sem, add=True)

            def pair_grp(p, _):
                g0 = 2 * p
                one_group(g0, staging, sem_a)

                @pl.when(g0 + 1 < ngrp)
                def _():
                    one_group(g0 + 1, staging_b, sem_b)

                return 0

            lax.fori_loop(0, (ngrp + 1) // 2, pair_grp, jnp.int32(0))

            @pl.when(ngrp >= 1)
            def _():
                pltpu.make_async_copy(staging, acc.at[iota], sem_a).wait()

            @pl.when(ngrp >= 2)
            def _():
                pltpu.make_async_copy(staging_b, acc.at[iota], sem_b).wait()

        total = sub_pass(jnp.int32(0))
        extract(jnp.minimum(total, _SCAP))

        def round_body(r, _):
            sub_pass(r)
            extract(jnp.minimum(total - r * _SCAP, _SCAP))
            return 0

        nrounds = (total + _SCAP - 1) // _SCAP
        lax.fori_loop(1, nrounds, round_body, jnp.int32(0))

    return process


@functools.lru_cache(maxsize=None)
def _make_scan_gather():
    mesh = plsc.VectorSubcoreMesh(core_axis_name="c", subcore_axis_name="s")

    @functools.partial(
        pl.kernel,
        mesh=mesh,
        out_type=jax.ShapeDtypeStruct((_NC, _AROWS, 128), jnp.float32),
        scratch_types=[
            pltpu.VMEM((_N,), jnp.int32),
            pltpu.VMEM((_HCAP,), jnp.int32),
            pltpu.VMEM((_HCAP,), jnp.int32),
            pltpu.VMEM((_SCAP + _L,), jnp.int32),
            pltpu.VMEM((_SCAP + _L,), jnp.int32),
            pltpu.VMEM((_L, 128), jnp.float32),
            pltpu.VMEM((_L, 128), jnp.float32),
            pltpu.VMEM((_D, _CW), jnp.float32),
            pltpu.VMEM((_D, _CW), jnp.float32),
            pltpu.VMEM_SHARED((_AROWS, 128), jnp.float32),
            pltpu.SemaphoreType.DMA,
            pltpu.SemaphoreType.DMA,
            pltpu.SemaphoreType.DMA,
            pltpu.SemaphoreType.DMA,
        ],
        compiler_params=pltpu.CompilerParams(
            use_tc_tiling_on_sc=True, needs_layout_passes=False),
    )
    def scan_kernel(idx_hbm, table_hbm, tail_hbm, out_hbm, idx_v, hits_i,
                    hits_n, subl_i, subl_n, staging, staging_b, buf0, buf1,
                    acc, sem0, sem1, sem_a, sem_b):
        scid = lax.axis_index("c")
        tid = lax.axis_index("s")
        wid = tid * _NC + scid

        zero = jnp.zeros((_L,), jnp.float32)
        for r in range(_L):
            for q in range(128 // _L):
                staging[r, pl.ds(q * _L, _L)] = zero
        rows_per_tile = _AROWS // _NS

        @pl.loop(0, rows_per_tile // _L)
        def _(r):
            pltpu.sync_copy(
                staging, acc.at[pl.ds(tid * rows_per_tile + r * _L, _L)])

        plsc.subcore_barrier()

        pltpu.sync_copy(idx_hbm, idx_v)
        lo = (wid * _NCHUNKS) >> 5
        hi = ((wid + 1) * _NCHUNKS) >> 5
        count = _filter_range(idx_v, hits_i, hits_n, lo * _CW, hi * _CW,
                              jnp.int32(_V - 1))

        process = _make_process(acc, subl_i, subl_n, hits_i, hits_n,
                                staging, staging_b, sem_a, sem_b, count)

        def issue(c, buf, sem):
            return pltpu.async_copy(
                table_hbm.at[:, pl.ds(pl.multiple_of(c * _CW, _CW), _CW)],
                buf, sem)

        def drain(sem):
            pltpu.make_async_copy(
                table_hbm.at[:, pl.ds(0, _CW)], buf0, sem).wait()

        nfull = jnp.minimum(hi, _NFULL) - lo

        @pl.when(nfull >= 1)
        def _():
            issue(lo, buf0, sem0)

        @pl.when(nfull >= 2)
        def _():
            issue(lo + 1, buf1, sem1)

        def pair_body(p, _):
            c = lo + 2 * p
            drain(sem0)
            process(buf0, c * _CW, _CW)

            @pl.when(c + 2 < lo + nfull)
            def _():
                issue(c + 2, buf0, sem0)

            drain(sem1)
            process(buf1, (c + 1) * _CW, _CW)

            @pl.when(c + 3 < lo + nfull)
            def _():
                issue(c + 3, buf1, sem1)

            return 0

        lax.fori_loop(0, nfull >> 1, pair_body, jnp.int32(0))

        @pl.when((nfull & 1) == 1)
        def _():
            drain(sem0)
            process(buf0, (lo + nfull - 1) * _CW, _CW)

        @pl.when(hi == _NCHUNKS)
        def _():
            pltpu.sync_copy(tail_hbm, buf0.at[:, pl.ds(0, _TAILP)])
            process(buf0, _NFULL * _CW, _TAILW)

        plsc.subcore_barrier()

        @pl.loop(0, 4)
        def _(r):
            rr = tid * rows_per_tile + r * (rows_per_tile // 4)
            pltpu.sync_copy(
                acc.at[pl.ds(rr, rows_per_tile // 4)],
                out_hbm.at[scid].at[pl.ds(rr, rows_per_tile // 4)])

    return scan_kernel


def kernel(batch_indices, batch_embeddings):
    num_rows = batch_embeddings.shape[0]
    idx = batch_indices.astype(jnp.int32)
    table_t = jnp.transpose(batch_embeddings.reshape(num_rows, _D))
    tail = jnp.pad(table_t[:, _NFULL * _CW:], ((0, 0), (0, _TAILP - _TAILW)))
    planes = _make_scan_gather()(idx, table_t, tail)
    out = planes[0] + planes[1]
    return out.reshape(_N, _D).reshape(_N, 1, _D)


# speedup vs baseline: 1.0168x; 1.0168x over previous
import functools

import jax
import jax.numpy as jnp
from jax import lax
from jax.experimental import pallas as pl
from jax.experimental.pallas import tpu as pltpu
from jax.experimental.pallas import tpu_sc as plsc

_N = 16384
_D = 32
_V = 1000000
_NC = 2
_NS = 16
_NW = _NC * _NS
_L = 16
_CW = 1024
_NFULL = _V // _CW
_TAILW = _V - _NFULL * _CW
_TAILP = 640
_NCHUNKS = _NFULL + 1
_HCAP = 4096 + _L
_SCAP = 2048
_AROWS = _N // 4


def _filter_range(idx_v, hits_i, hits_n, lo_col, hi_col, max_row):
    iota = lax.iota(jnp.int32, _L)

    def body(k, pos):
        iv = jnp.minimum(idx_v[pl.ds(k * _L, _L)], max_row)
        nv = iota + k * _L
        m = (iv >= lo_col) & (iv < hi_col)
        mi = m.astype(jnp.int32)
        pfx = plsc.cumsum(mi)
        dest = pos + pfx - mi
        ms = m & (dest < _HCAP - _L)
        plsc.store_scatter(hits_i, [dest], iv, mask=ms)
        plsc.store_scatter(hits_n, [dest], nv, mask=ms)
        return pos + plsc.all_reduce_population_count(m)[0]

    return lax.fori_loop(0, _N // _L, body, jnp.int32(0), unroll=8)


def _make_process(acc, subl_i, subl_n, hits_i, hits_n, staging, count):
    iota = lax.iota(jnp.int32, _L)

    def process(buf, c0, cw):
        nh = (count + _L - 1) // _L

        def sub_pass(r):
            base = r * _SCAP

            def sub_body(j, spos):
                valid = iota < (count - j * _L)
                iv = hits_i[pl.ds(j * _L, _L)]
                nv = hits_n[pl.ds(j * _L, _L)]
                m = valid & (iv >= c0) & (iv < c0 + cw)
                mi = m.astype(jnp.int32)
                pfx = plsc.cumsum(mi)
                dest = spos + pfx - mi - base
                ms = m & (dest >= 0) & (dest < _SCAP)
                dclip = jnp.where(ms, dest, 0)
                plsc.store_scatter(subl_i, [dclip], iv, mask=ms)
                plsc.store_scatter(subl_n, [dclip], nv, mask=ms)
                return spos + plsc.all_reduce_population_count(m)[0]

            return lax.fori_loop(0, nh, sub_body, jnp.int32(0))

        def extract(nsel):
            def grp_body(g, _):
                m = iota < (nsel - g * _L)
                iv = subl_i[pl.ds(g * _L, _L)]
                nv = subl_n[pl.ds(g * _L, _L)]
                colv = jnp.where(m, iv - c0, 0)
                seg = jnp.where(m, (nv & 3) << 5, 0)
                jrow = jnp.where(m, nv >> 2, 0)
                zero = jnp.zeros((_L,), jnp.float32)
                for r in range(_L):
                    for q in range(128 // _L):
                        staging[r, pl.ds(q * _L, _L)] = zero
                for d in range(_D):
                    vals = plsc.load_gather(
                        buf, [jnp.full((_L,), d, jnp.int32), colv], mask=m)
                    plsc.store_scatter(staging, [iota, seg + d], vals,
                                       mask=m)
                pltpu.sync_copy(staging, acc.at[jrow], add=True)
                return 0

            ngrp = (nsel + _L - 1) // _L
            lax.fori_loop(0, ngrp, grp_body, jnp.int32(0))

        total = sub_pass(jnp.int32(0))
        extract(jnp.minimum(total, _SCAP))

        def round_body(r, _):
            sub_pass(r)
            extract(jnp.minimum(total - r * _SCAP, _SCAP))
            return 0

        nrounds = (total + _SCAP - 1) // _SCAP
        lax.fori_loop(1, nrounds, round_body, jnp.int32(0))

    return process


@functools.lru_cache(maxsize=None)
def _make_scan_gather():
    mesh = plsc.VectorSubcoreMesh(core_axis_name="c", subcore_axis_name="s")

    @functools.partial(
        pl.kernel,
        mesh=mesh,
        out_type=jax.ShapeDtypeStruct((_NC, _AROWS, 128), jnp.float32),
        scratch_types=[
            pltpu.VMEM((_N,), jnp.int32),
            pltpu.VMEM((_HCAP,), jnp.int32),
            pltpu.VMEM((_HCAP,), jnp.int32),
            pltpu.VMEM((_SCAP + _L,), jnp.int32),
            pltpu.VMEM((_SCAP + _L,), jnp.int32),
            pltpu.VMEM((_L, 128), jnp.float32),
            pltpu.VMEM((_D, _CW), jnp.float32),
            pltpu.VMEM((_D, _CW), jnp.float32),
            pltpu.VMEM_SHARED((_AROWS, 128), jnp.float32),
            pltpu.SemaphoreType.DMA,
            pltpu.SemaphoreType.DMA,
        ],
        compiler_params=pltpu.CompilerParams(
            use_tc_tiling_on_sc=True, needs_layout_passes=False),
    )
    def scan_kernel(idx_hbm, table_hbm, tail_hbm, out_hbm, idx_v, hits_i,
                    hits_n, subl_i, subl_n, staging, buf0, buf1, acc,
                    sem0, sem1):
        scid = lax.axis_index("c")
        tid = lax.axis_index("s")
        wid = tid * _NC + scid

        zero = jnp.zeros((_L,), jnp.float32)
        for r in range(_L):
            for q in range(128 // _L):
                staging[r, pl.ds(q * _L, _L)] = zero
        rows_per_tile = _AROWS // _NS

        @pl.loop(0, rows_per_tile // _L)
        def _(r):
            pltpu.sync_copy(
                staging, acc.at[pl.ds(tid * rows_per_tile + r * _L, _L)])

        plsc.subcore_barrier()

        pltpu.sync_copy(idx_hbm, idx_v)
        lo = (wid * _NCHUNKS) >> 5
        hi = ((wid + 1) * _NCHUNKS) >> 5
        count = _filter_range(idx_v, hits_i, hits_n, lo * _CW, hi * _CW,
                              jnp.int32(_V - 1))

        process = _make_process(acc, subl_i, subl_n, hits_i, hits_n,
                                staging, count)

        def issue(c, buf, sem):
            return pltpu.async_copy(
                table_hbm.at[:, pl.ds(pl.multiple_of(c * _CW, _CW), _CW)],
                buf, sem)

        def drain(sem):
            pltpu.make_async_copy(
                table_hbm.at[:, pl.ds(0, _CW)], buf0, sem).wait()

        nfull = jnp.minimum(hi, _NFULL) - lo

        @pl.when(nfull >= 1)
        def _():
            issue(lo, buf0, sem0)

        @pl.when(nfull >= 2)
        def _():
            issue(lo + 1, buf1, sem1)

        def pair_body(p, _):
            c = lo + 2 * p
            drain(sem0)
            process(buf0, c * _CW, _CW)

            @pl.when(c + 2 < lo + nfull)
            def _():
                issue(c + 2, buf0, sem0)

            drain(sem1)
            process(buf1, (c + 1) * _CW, _CW)

            @pl.when(c + 3 < lo + nfull)
            def _():
                issue(c + 3, buf1, sem1)

            return 0

        lax.fori_loop(0, nfull >> 1, pair_body, jnp.int32(0))

        @pl.when((nfull & 1) == 1)
        def _():
            drain(sem0)
            process(buf0, (lo + nfull - 1) * _CW, _CW)

        @pl.when(hi == _NCHUNKS)
        def _():
            pltpu.sync_copy(tail_hbm, buf0.at[:, pl.ds(0, _TAILP)])
            process(buf0, _NFULL * _CW, _TAILW)

        plsc.subcore_barrier()

        @pl.loop(0, 4)
        def _(r):
            rr = tid * rows_per_tile + r * (rows_per_tile // 4)
            pltpu.sync_copy(
                acc.at[pl.ds(rr, rows_per_tile // 4)],
                out_hbm.at[scid].at[pl.ds(rr, rows_per_tile // 4)])

    return scan_kernel


def kernel(batch_indices, batch_embeddings):
    num_rows = batch_embeddings.shape[0]
    idx = batch_indices.astype(jnp.int32)
    table_t = jnp.transpose(batch_embeddings.reshape(num_rows, _D))
    tail = jnp.pad(table_t[:, _NFULL * _CW:], ((0, 0), (0, _TAILP - _TAILW)))
    planes = _make_scan_gather()(idx, table_t, tail)
    out = planes[0] + planes[1]
    return out.reshape(_N, _D).reshape(_N, 1, _D)
